# baseline (speedup 1.0000x reference)
import functools

import jax
import jax.numpy as jnp
from jax.experimental import pallas as pl
from jax.experimental.pallas import tpu as pltpu


def _dhcf_kernel(adj_ref, u_ref, i_ref, hu_ref, hi_ref, tu_ref, ti_ref,
                 *, bu, bi_sz, nbu, nbi, leaky):
    p = pl.program_id(0)
    bi = pl.program_id(1)
    bj = pl.program_id(2)

    first = (bi == 0) & (bj == 0)
    last = (bi == nbu - 1) & (bj == nbi - 1)

    @pl.when((p == 0) & first)
    def _init():
        tu_ref[...] = jnp.zeros_like(tu_ref)
        ti_ref[...] = jnp.zeros_like(ti_ref)
        hu_ref[...] = jnp.zeros_like(hu_ref)
        hi_ref[...] = jnp.zeros_like(hi_ref)

    a = adj_ref[...].astype(jnp.bfloat16)

    @pl.when(p == 0)
    def _phase0():
        u_blk = u_ref[pl.ds(bi * bu, bu), :].astype(jnp.bfloat16)
        i_blk = i_ref[pl.ds(bj * bi_sz, bi_sz), :].astype(jnp.bfloat16)
        tu_ref[pl.ds(bj * bi_sz, bi_sz), :] += jax.lax.dot_general(
            a, u_blk, (((0,), (0,)), ((), ())),
            preferred_element_type=jnp.float32)
        ti_ref[pl.ds(bi * bu, bu), :] += jax.lax.dot_general(
            a, i_blk, (((1,), (0,)), ((), ())),
            preferred_element_type=jnp.float32)

    @pl.when(p == 1)
    def _phase1():
        tu_blk = tu_ref[pl.ds(bj * bi_sz, bi_sz), :].astype(jnp.bfloat16)
        ti_blk = ti_ref[pl.ds(bi * bu, bu), :].astype(jnp.bfloat16)
        hu_ref[pl.ds(bi * bu, bu), :] += jax.lax.dot_general(
            a, tu_blk, (((1,), (0,)), ((), ())),
            preferred_element_type=jnp.float32)
        hi_ref[pl.ds(bj * bi_sz, bi_sz), :] += jax.lax.dot_general(
            a, ti_blk, (((0,), (0,)), ((), ())),
            preferred_element_type=jnp.float32)

    @pl.when((p == 1) & last)
    def _act():
        hu = hu_ref[...]
        hu_ref[...] = jnp.where(hu >= 0, hu, leaky * hu)
        hi = hi_ref[...]
        hi_ref[...] = jnp.where(hi >= 0, hi, leaky * hi)


@jax.jit
def kernel(adj, user_emb, item_emb):
    n_users, n_items = adj.shape
    hd = user_emb.shape[1]
    bu = min(n_users, 512)
    bi_sz = min(n_items, 512)
    nbu = n_users // bu
    nbi = n_items // bi_sz

    body = functools.partial(_dhcf_kernel, bu=bu, bi_sz=bi_sz,
                             nbu=nbu, nbi=nbi, leaky=0.5)
    h_u, h_i = pl.pallas_call(
        body,
        grid=(2, nbu, nbi),
        in_specs=[
            pl.BlockSpec((bu, bi_sz), lambda p, i, j: (i, j)),
            pl.BlockSpec((n_users, hd), lambda p, i, j: (0, 0)),
            pl.BlockSpec((n_items, hd), lambda p, i, j: (0, 0)),
        ],
        out_specs=[
            pl.BlockSpec((n_users, hd), lambda p, i, j: (0, 0)),
            pl.BlockSpec((n_items, hd), lambda p, i, j: (0, 0)),
        ],
        out_shape=[
            jax.ShapeDtypeStruct((n_users, hd), jnp.float32),
            jax.ShapeDtypeStruct((n_items, hd), jnp.float32),
        ],
        scratch_shapes=[
            pltpu.VMEM((n_items, hd), jnp.float32),
            pltpu.VMEM((n_users, hd), jnp.float32),
        ],
    )(adj, user_emb, item_emb)

    user_all = jnp.concatenate([user_emb, h_u, h_u], axis=1)
    item_all = jnp.concatenate([item_emb, h_i, h_i], axis=1)
    return (user_all, item_all)
